# baseline (speedup 1.0000x reference)
import jax
import jax.numpy as jnp
from jax.experimental import pallas as pl
from jax.experimental.pallas import tpu as pltpu


def _wide_kernel(wb_ref, x_ref, o_ref):
    wb = wb_ref[...]
    k = x_ref.shape[0]
    o_ref[...] = (
        jnp.dot(wb[:, :k], x_ref[...], preferred_element_type=jnp.float32)
        + wb[:, k : k + 1]
    )


def kernel(feat, server_model, len_seq, mask, manfeat, emb1_w, emb2_w, emb3_w,
           emb4_w, emb5_w, k_w, o_w, cls_w, cls_b, wide_w, wide_b):
    b, k = manfeat.shape
    n = wide_w.shape[1]
    xt = manfeat.T
    wbt = jnp.concatenate([wide_w, wide_b[None, :]], axis=0).T
    out_t = pl.pallas_call(
        _wide_kernel,
        out_shape=jax.ShapeDtypeStruct((n, b), jnp.float32),
        compiler_params=pltpu.CompilerParams(
            disable_bounds_checks=True,
            disable_semaphore_checks=True,
        ),
    )(wbt, xt)
    return out_t.T
